# baseline (speedup 1.0000x reference)
import functools

import jax
import jax.numpy as jnp
from jax import lax
from jax.experimental import pallas as pl
from jax.experimental.pallas import tpu as pltpu
from jax.experimental.pallas import tpu_sc as plsc

_TEMP = 0.05
_B = 4096
_D = 32
_DP = 128

_NC, _NS = 2, 16
_NW = _NC * _NS
_BPW = _B // _NW


@functools.cache
def _make_sc_gather():
    mesh = plsc.VectorSubcoreMesh(core_axis_name="c", subcore_axis_name="s")

    def _widen(rows_v, wide_v):
        for r in range(_BPW):
            for c in range(_D // 16):
                wide_v[r, pl.ds(c * 16, 16)] = rows_v[r, pl.ds(c * 16, 16)]

    @functools.partial(
        pl.kernel,
        mesh=mesh,
        out_type=[
            jax.ShapeDtypeStruct((_B, _D), jnp.float32),
            jax.ShapeDtypeStruct((_B, _D), jnp.float32),
        ],
        scratch_types=[
            pltpu.VMEM((_BPW,), jnp.int32),
            pltpu.VMEM((_BPW, _D), jnp.float32),
            pltpu.VMEM((_BPW, _DP), jnp.float32),
            pltpu.VMEM((_BPW,), jnp.int32),
            pltpu.VMEM((_BPW, _D), jnp.float32),
            pltpu.VMEM((_BPW, _DP), jnp.float32),
            pltpu.SemaphoreType.DMA,
            pltpu.SemaphoreType.DMA,
        ],
        compiler_params=pltpu.CompilerParams(
            use_tc_tiling_on_sc=False,
            disable_bounds_checks=True,
            disable_semaphore_checks=True,
        ),
    )
    def _sc_gather(u_ids_hbm, i_ids_hbm, u_table_hbm, i_table_hbm,
                   u_out, i_out, u_idx_v, u_rows_v, u_wide_v,
                   i_idx_v, i_rows_v, i_wide_v, u_sem, i_sem):
        wid = lax.axis_index("s") * _NC + lax.axis_index("c")
        base = wid * _BPW
        u_icp = pltpu.async_copy(u_ids_hbm.at[pl.ds(base, _BPW)], u_idx_v, u_sem)
        i_icp = pltpu.async_copy(i_ids_hbm.at[pl.ds(base, _BPW)], i_idx_v, i_sem)
        u_icp.wait()
        u_cp = pltpu.async_copy(u_table_hbm.at[u_idx_v], u_rows_v, u_sem)
        i_icp.wait()
        i_cp = pltpu.async_copy(i_table_hbm.at[i_idx_v], i_rows_v, i_sem)
        u_cp.wait()
        _widen(u_rows_v, u_wide_v)
        u_ocp = pltpu.async_copy(u_rows_v, u_out.at[pl.ds(base, _BPW)], u_sem)
        i_cp.wait()
        _widen(i_rows_v, i_wide_v)
        i_ocp = pltpu.async_copy(i_rows_v, i_out.at[pl.ds(base, _BPW)], i_sem)
        u_ocp.wait()
        i_ocp.wait()

    return _sc_gather


_TM = 512


def _lane_mask(x):
    lane = lax.broadcasted_iota(jnp.int32, x.shape, 1)
    return jnp.where(lane < _D, x, 0.0)


def _tc_dot_body(g_ref, h_ref, out_ref, hn_ref):
    @pl.when(pl.program_id(0) == 0)
    def _():
        h = _lane_mask(h_ref[...])
        sh = jnp.sum(h * h, axis=1, keepdims=True)
        hn_ref[...] = (h * lax.rsqrt(jnp.maximum(sh, 1e-24))
                       ).astype(jnp.bfloat16)

    g = _lane_mask(g_ref[...])
    sg = jnp.sum(g * g, axis=1, keepdims=True)
    gn = (g * ((1.0 / _TEMP) * lax.rsqrt(jnp.maximum(sg, 1e-24)))
          ).astype(jnp.bfloat16)
    out_ref[...] = lax.dot_general(
        gn, hn_ref[...], (((1,), (1,)), ((), ())),
        preferred_element_type=jnp.float32)


def _tc_matmul(g, h):
    return pl.pallas_call(
        _tc_dot_body,
        grid=(_B // _TM,),
        in_specs=[
            pl.BlockSpec((_TM, _DP), lambda b: (b, 0)),
            pl.BlockSpec((_B, _DP), lambda b: (0, 0)),
        ],
        out_specs=pl.BlockSpec((_TM, _B), lambda b: (b, 0)),
        out_shape=jax.ShapeDtypeStruct((_B, _B), jnp.float32),
        scratch_shapes=[pltpu.VMEM((_B, _DP), jnp.bfloat16)],
    )(g, h)


def kernel(u_ids, i_ids, u_table, i_table):
    g, h = _make_sc_gather()(u_ids, i_ids, u_table, i_table)
    gp = jnp.pad(g, ((0, 0), (0, _DP - _D)))
    hp = jnp.pad(h, ((0, 0), (0, _DP - _D)))
    return _tc_matmul(gp, hp)


# speedup vs baseline: 1.1114x; 1.1114x over previous
import functools

import jax
import jax.numpy as jnp
from jax import lax
from jax.experimental import pallas as pl
from jax.experimental.pallas import tpu as pltpu
from jax.experimental.pallas import tpu_sc as plsc

_TEMP = 0.05
_B = 4096
_D = 32
_DP = 128
_R = _DP // _D

_NC, _NS = 2, 16
_NW = _NC * _NS
_BPW = _B // _NW


@functools.cache
def _make_sc_gather():
    mesh = plsc.VectorSubcoreMesh(core_axis_name="c", subcore_axis_name="s")

    @functools.partial(
        pl.kernel,
        mesh=mesh,
        out_type=[
            jax.ShapeDtypeStruct((_R * _B, _D), jnp.float32),
            jax.ShapeDtypeStruct((_R * _B, _D), jnp.float32),
        ],
        scratch_types=[
            pltpu.VMEM((_BPW,), jnp.int32),
            pltpu.VMEM((_BPW,), jnp.int32),
            pltpu.VMEM((_BPW, _D), jnp.float32),
            pltpu.VMEM((_BPW,), jnp.int32),
            pltpu.VMEM((_BPW, _D), jnp.float32),
            pltpu.SemaphoreType.DMA,
            pltpu.SemaphoreType.DMA,
        ],
        compiler_params=pltpu.CompilerParams(
            use_tc_tiling_on_sc=False,
            disable_bounds_checks=True,
            disable_semaphore_checks=True,
        ),
    )
    def _sc_gather(u_ids_hbm, i_ids_hbm, u_table_hbm, i_table_hbm,
                   u_out, i_out, sidx_v, u_idx_v, u_rows_v,
                   i_idx_v, i_rows_v, u_sem, i_sem):
        wid = lax.axis_index("s") * _NC + lax.axis_index("c")
        base = wid * _BPW
        u_icp = pltpu.async_copy(u_ids_hbm.at[pl.ds(base, _BPW)], u_idx_v, u_sem)
        i_icp = pltpu.async_copy(i_ids_hbm.at[pl.ds(base, _BPW)], i_idx_v, i_sem)
        for j in range(_BPW // 16):
            sidx_v[pl.ds(j * 16, 16)] = (
                (base + j * 16 + jnp.arange(16, dtype=jnp.int32)) * _R)
        u_icp.wait()
        u_cp = pltpu.async_copy(u_table_hbm.at[u_idx_v], u_rows_v, u_sem)
        i_icp.wait()
        i_cp = pltpu.async_copy(i_table_hbm.at[i_idx_v], i_rows_v, i_sem)
        u_cp.wait()
        u_ocp = pltpu.async_copy(u_rows_v, u_out.at[sidx_v], u_sem)
        i_cp.wait()
        i_ocp = pltpu.async_copy(i_rows_v, i_out.at[sidx_v], i_sem)
        u_ocp.wait()
        i_ocp.wait()

    return _sc_gather


_TM = 512


def _lane_mask(x):
    lane = lax.broadcasted_iota(jnp.int32, x.shape, 1)
    return jnp.where(lane < _D, x, 0.0)


def _tc_dot_body(g_ref, h_ref, out_ref, hn_ref):
    @pl.when(pl.program_id(0) == 0)
    def _():
        h = _lane_mask(h_ref[...])
        sh = jnp.sum(h * h, axis=1, keepdims=True)
        hn_ref[...] = (h * lax.rsqrt(jnp.maximum(sh, 1e-24))
                       ).astype(jnp.bfloat16)

    g = _lane_mask(g_ref[...])
    sg = jnp.sum(g * g, axis=1, keepdims=True)
    gn = (g * ((1.0 / _TEMP) * lax.rsqrt(jnp.maximum(sg, 1e-24)))
          ).astype(jnp.bfloat16)
    out_ref[...] = lax.dot_general(
        gn, hn_ref[...], (((1,), (1,)), ((), ())),
        preferred_element_type=jnp.float32)


def _tc_matmul(g, h):
    return pl.pallas_call(
        _tc_dot_body,
        grid=(_B // _TM,),
        in_specs=[
            pl.BlockSpec((_TM, _DP), lambda b: (b, 0)),
            pl.BlockSpec((_B, _DP), lambda b: (0, 0)),
        ],
        out_specs=pl.BlockSpec((_TM, _B), lambda b: (b, 0)),
        out_shape=jax.ShapeDtypeStruct((_B, _B), jnp.float32),
        scratch_shapes=[pltpu.VMEM((_B, _DP), jnp.bfloat16)],
    )(g, h)


def kernel(u_ids, i_ids, u_table, i_table):
    g4, h4 = _make_sc_gather()(u_ids, i_ids, u_table, i_table)
    g = g4.reshape(_B, _DP)
    h = h4.reshape(_B, _DP)
    return _tc_matmul(g, h)
